# baseline (speedup 1.0000x reference)
import jax
import jax.numpy as jnp
from jax.experimental import pallas as pl
from jax.experimental.pallas import tpu as pltpu

S, D, H = 2048, 768, 12
DH = D // H
W1 = 256
BQ = 256
NB = S // BQ


def _qkv_kernel(h_ref, wq_ref, wk_ref, wv_ref, bq_ref, bk_ref, bv_ref,
                q_ref, k_ref, v_ref):
    h = h_ref[...]
    scale = jnp.float32(1.0 / 8.0)
    q = (jnp.dot(h, wq_ref[...], preferred_element_type=jnp.float32)
         + bq_ref[...]) * scale
    q_ref[...] = q.astype(jnp.bfloat16)
    k = jnp.dot(h, wk_ref[...], preferred_element_type=jnp.float32) + bk_ref[...]
    k_ref[...] = k.astype(jnp.bfloat16)
    v = jnp.dot(h, wv_ref[...], preferred_element_type=jnp.float32) + bv_ref[...]
    v_ref[...] = v.astype(jnp.bfloat16)


def _attn_kernel(q_ref, kp_ref, kc_ref, kn_ref, vp_ref, vc_ref, vn_ref,
                 mp_ref, mc_ref, mn_ref, qm_ref, wo_ref, bo_ref, out_ref):
    qi = pl.program_id(0)
    q = q_ref[...]
    K = jnp.concatenate([kp_ref[...], kc_ref[...], kn_ref[...]], axis=0)
    V = jnp.concatenate([vp_ref[...], vc_ref[...], vn_ref[...]], axis=0)
    am = jnp.concatenate([mp_ref[...], mc_ref[...], mn_ref[...]], axis=1)
    row = jax.lax.broadcasted_iota(jnp.int32, (BQ, 3 * BQ), 0)
    col = jax.lax.broadcasted_iota(jnp.int32, (BQ, 3 * BQ), 1)
    valid = (col >= row) & (col <= row + 2 * W1)
    valid &= (col >= BQ) | (qi > 0)
    valid &= (col < 2 * BQ) | (qi < NB - 1)
    madd = jnp.where(valid, am, jnp.float32(-1e9))
    ctx_parts = []
    for h in range(H):
        sl = slice(h * DH, (h + 1) * DH)
        s = jax.lax.dot_general(q[:, sl], K[:, sl], (((1,), (1,)), ((), ())),
                                preferred_element_type=jnp.float32)
        s = s + madd
        m = jnp.max(s, axis=1, keepdims=True)
        e = jnp.exp(s - m)
        r = 1.0 / jnp.sum(e, axis=1, keepdims=True)
        pv = jnp.dot(e.astype(jnp.bfloat16), V[:, sl],
                     preferred_element_type=jnp.float32)
        ctx_parts.append(pv * r)
    ctx = jnp.concatenate(ctx_parts, axis=1) * qm_ref[...]
    out_ref[...] = jnp.dot(ctx.astype(jnp.bfloat16), wo_ref[...],
                           preferred_element_type=jnp.float32) + bo_ref[...]


def _run(hs, am, qm, Wq, Wk, Wv, bq, bk, bv, Wo, bo, interpret=False):
    bf = jnp.bfloat16
    q, k, v = pl.pallas_call(
        _qkv_kernel,
        grid=(NB,),
        in_specs=[
            pl.BlockSpec((BQ, D), lambda i: (i, 0)),
            pl.BlockSpec((D, D), lambda i: (0, 0)),
            pl.BlockSpec((D, D), lambda i: (0, 0)),
            pl.BlockSpec((D, D), lambda i: (0, 0)),
            pl.BlockSpec((1, D), lambda i: (0, 0)),
            pl.BlockSpec((1, D), lambda i: (0, 0)),
            pl.BlockSpec((1, D), lambda i: (0, 0)),
        ],
        out_specs=[pl.BlockSpec((BQ, D), lambda i: (i, 0))] * 3,
        out_shape=[jax.ShapeDtypeStruct((S, D), bf)] * 3,
        compiler_params=pltpu.CompilerParams(
            dimension_semantics=("arbitrary",)),
        interpret=interpret,
    )(hs.astype(bf), Wq.astype(bf), Wk.astype(bf), Wv.astype(bf), bq, bk, bv)

    prev = lambda i: jnp.maximum(i - 1, 0)
    nxt = lambda i: jnp.minimum(i + 1, NB - 1)
    out = pl.pallas_call(
        _attn_kernel,
        grid=(NB,),
        in_specs=[
            pl.BlockSpec((BQ, D), lambda i: (i, 0)),
            pl.BlockSpec((BQ, D), lambda i: (prev(i), 0)),
            pl.BlockSpec((BQ, D), lambda i: (i, 0)),
            pl.BlockSpec((BQ, D), lambda i: (nxt(i), 0)),
            pl.BlockSpec((BQ, D), lambda i: (prev(i), 0)),
            pl.BlockSpec((BQ, D), lambda i: (i, 0)),
            pl.BlockSpec((BQ, D), lambda i: (nxt(i), 0)),
            pl.BlockSpec((1, BQ), lambda i: (0, prev(i))),
            pl.BlockSpec((1, BQ), lambda i: (0, i)),
            pl.BlockSpec((1, BQ), lambda i: (0, nxt(i))),
            pl.BlockSpec((BQ, 1), lambda i: (i, 0)),
            pl.BlockSpec((D, D), lambda i: (0, 0)),
            pl.BlockSpec((1, D), lambda i: (0, 0)),
        ],
        out_specs=pl.BlockSpec((BQ, D), lambda i: (i, 0)),
        out_shape=jax.ShapeDtypeStruct((S, D), jnp.float32),
        compiler_params=pltpu.CompilerParams(
            dimension_semantics=("arbitrary",)),
        interpret=interpret,
    )(q, k, k, k, v, v, v, am, am, am, qm, Wo.astype(bf), bo)
    return out


def kernel(hidden_states, attention_mask, Wq, bq, Wk, bk, Wv, bv, Wo, bo,
           is_index_masked, is_index_global_attn, is_global_attn):
    b, s, d = hidden_states.shape
    hs = hidden_states.reshape(s, d)
    am = attention_mask.reshape(1, s).astype(jnp.float32)
    qm = (1.0 - is_index_masked.reshape(s).astype(jnp.float32))[:, None]
    out = _run(hs, am, qm, Wq, Wk, Wv,
               bq[None, :], bk[None, :], bv[None, :], Wo, bo[None, :])
    return out.reshape(b, s, d)


# speedup vs baseline: 1.0729x; 1.0729x over previous
import jax
import jax.numpy as jnp
from jax.experimental import pallas as pl
from jax.experimental.pallas import tpu as pltpu

S, D, H = 2048, 768, 12
DH = D // H
W1 = 256
BQ = 256
NB = S // BQ


def _fused_kernel(h_ref, wq_ref, wk_ref, wv_ref, bq_ref, bk_ref, bv_ref,
                  mp_ref, mc_ref, mn_ref, qm_ref, wo_ref, bo_ref, out_ref,
                  qs, ks, vs):
    j = pl.program_id(0)

    @pl.when(j < NB)
    def _proj():
        h = h_ref[...]
        base = j * BQ
        q = (jnp.dot(h, wq_ref[...], preferred_element_type=jnp.float32)
             + bq_ref[...]) * jnp.float32(1.0 / 8.0)
        qs[pl.ds(base, BQ), :] = q.astype(jnp.bfloat16)
        k = jnp.dot(h, wk_ref[...], preferred_element_type=jnp.float32) + bk_ref[...]
        ks[pl.ds(base, BQ), :] = k.astype(jnp.bfloat16)
        v = jnp.dot(h, wv_ref[...], preferred_element_type=jnp.float32) + bv_ref[...]
        vs[pl.ds(base, BQ), :] = v.astype(jnp.bfloat16)

    @pl.when(j > 0)
    def _attn():
        i = j - 1
        bp = jnp.maximum(i - 1, 0)
        bn = jnp.minimum(i + 1, NB - 1)
        q = qs[pl.ds(i * BQ, BQ), :]
        K = jnp.concatenate([ks[pl.ds(bp * BQ, BQ), :],
                             ks[pl.ds(i * BQ, BQ), :],
                             ks[pl.ds(bn * BQ, BQ), :]], axis=0)
        V = jnp.concatenate([vs[pl.ds(bp * BQ, BQ), :],
                             vs[pl.ds(i * BQ, BQ), :],
                             vs[pl.ds(bn * BQ, BQ), :]], axis=0)
        am = jnp.concatenate([mp_ref[...], mc_ref[...], mn_ref[...]], axis=1)
        row = jax.lax.broadcasted_iota(jnp.int32, (BQ, 3 * BQ), 0)
        col = jax.lax.broadcasted_iota(jnp.int32, (BQ, 3 * BQ), 1)
        valid = (col >= row) & (col <= row + 2 * W1)
        valid &= (col >= BQ) | (i > 0)
        valid &= (col < 2 * BQ) | (i < NB - 1)
        madd = jnp.where(valid, am, jnp.float32(-1e9))
        ctx_parts = []
        for h in range(H):
            sl = slice(h * DH, (h + 1) * DH)
            s = jax.lax.dot_general(q[:, sl], K[:, sl],
                                    (((1,), (1,)), ((), ())),
                                    preferred_element_type=jnp.float32)
            s = s + madd
            m = jnp.max(s, axis=1, keepdims=True)
            e = jnp.exp(s - m)
            r = 1.0 / jnp.sum(e, axis=1, keepdims=True)
            pv = jnp.dot(e.astype(jnp.bfloat16), V[:, sl],
                         preferred_element_type=jnp.float32)
            ctx_parts.append(pv * r)
        ctx = jnp.concatenate(ctx_parts, axis=1) * qm_ref[...]
        out_ref[...] = jnp.dot(ctx.astype(jnp.bfloat16), wo_ref[...],
                               preferred_element_type=jnp.float32) + bo_ref[...]


def _run(hs, am, qm, Wq, Wk, Wv, bq, bk, bv, Wo, bo, interpret=False):
    cur = lambda j: jnp.maximum(j - 1, 0)
    prev = lambda j: jnp.maximum(j - 2, 0)
    nxt = lambda j: jnp.minimum(jnp.maximum(j, 1), NB - 1)
    out = pl.pallas_call(
        _fused_kernel,
        grid=(NB + 1,),
        in_specs=[
            pl.BlockSpec((BQ, D), lambda j: (jnp.minimum(j, NB - 1), 0)),
            pl.BlockSpec((D, D), lambda j: (0, 0)),
            pl.BlockSpec((D, D), lambda j: (0, 0)),
            pl.BlockSpec((D, D), lambda j: (0, 0)),
            pl.BlockSpec((1, D), lambda j: (0, 0)),
            pl.BlockSpec((1, D), lambda j: (0, 0)),
            pl.BlockSpec((1, D), lambda j: (0, 0)),
            pl.BlockSpec((1, BQ), lambda j: (0, prev(j))),
            pl.BlockSpec((1, BQ), lambda j: (0, cur(j))),
            pl.BlockSpec((1, BQ), lambda j: (0, nxt(j))),
            pl.BlockSpec((BQ, 1), lambda j: (cur(j), 0)),
            pl.BlockSpec((D, D), lambda j: (0, 0)),
            pl.BlockSpec((1, D), lambda j: (0, 0)),
        ],
        out_specs=pl.BlockSpec((BQ, D), lambda j: (cur(j), 0)),
        out_shape=jax.ShapeDtypeStruct((S, D), jnp.float32),
        scratch_shapes=[
            pltpu.VMEM((S, D), jnp.bfloat16),
            pltpu.VMEM((S, D), jnp.bfloat16),
            pltpu.VMEM((S, D), jnp.bfloat16),
        ],
        compiler_params=pltpu.CompilerParams(
            dimension_semantics=("arbitrary",)),
        interpret=interpret,
    )(hs, Wq, Wk, Wv, bq, bk, bv, am, am, am, qm, Wo.astype(jnp.bfloat16), bo)
    return out


def kernel(hidden_states, attention_mask, Wq, bq, Wk, bk, Wv, bv, Wo, bo,
           is_index_masked, is_index_global_attn, is_global_attn):
    b, s, d = hidden_states.shape
    hs = hidden_states.reshape(s, d)
    am = attention_mask.reshape(1, s).astype(jnp.float32)
    qm = (1.0 - is_index_masked.reshape(s).astype(jnp.float32))[:, None]
    out = _run(hs, am, qm, Wq, Wk, Wv,
               bq[None, :], bk[None, :], bv[None, :], Wo, bo[None, :])
    return out.reshape(b, s, d)
